# Initial kernel scaffold; baseline (speedup 1.0000x reference)
import jax
import jax.numpy as jnp
from jax.experimental import pallas as pl


def kernel(feat_rows, feat_cols, feat_vals, edge_index, adj_vals, weights):
    raise NotImplementedError("write your pallas kernel here")


# speedup vs baseline: 3.4392x; 3.4392x over previous
import dataclasses
import functools

import jax
import jax.numpy as jnp
from jax import lax
from jax.experimental import pallas as pl
from jax.experimental.pallas import tpu as pltpu
from jax.experimental.pallas import tpu_sc as plsc

N = 10000
D = 128
NNZ = 320000

NUM_CORES = 2
NUM_SUBCORES = 16
NW = NUM_CORES * NUM_SUBCORES
PER_W = NNZ // NW
CH = 80
N_CHUNKS = PER_W // CH
ZCH = 80
N_ZCH = N // ZCH


def _sc_phase(table_rows: int):
    mesh = plsc.VectorSubcoreMesh(core_axis_name="c", subcore_axis_name="s")
    cp = pltpu.CompilerParams()
    if "needs_layout_passes" in pltpu.CompilerParams.__dataclass_fields__:
        cp = dataclasses.replace(cp, needs_layout_passes=False)

    @functools.partial(
        pl.kernel,
        compiler_params=cp,
        out_type=jax.ShapeDtypeStruct((NUM_CORES, N, D), jnp.float32),
        mesh=mesh,
        scratch_types=[
            pltpu.VMEM_SHARED((N, D), jnp.float32),
            pltpu.VMEM((CH,), jnp.int32),
            pltpu.VMEM((CH,), jnp.int32),
            pltpu.VMEM((CH,), jnp.float32),
            pltpu.VMEM((CH, D), jnp.float32),
            pltpu.SemaphoreType.DMA,
        ],
    )
    def phase(table_hbm, gidx_hbm, sidx_hbm, vals_hbm, out_hbm,
              acc, gi_v, si_v, va_v, rows_v, sem):
        cid = lax.axis_index("c")
        sid = lax.axis_index("s")
        wid = cid * NUM_SUBCORES + sid

        zero16 = jnp.zeros((16,), jnp.float32)

        @pl.loop(0, CH)
        def _(j):
            for g in range(D // 16):
                rows_v[j, pl.ds(g * 16, 16)] = zero16

        @pl.loop(sid, N_ZCH, step=NUM_SUBCORES)
        def _(k):
            base = pl.multiple_of(k * ZCH, ZCH)
            pltpu.sync_copy(rows_v, acc.at[pl.ds(base, ZCH), :])

        plsc.subcore_barrier()

        @pl.loop(0, N_CHUNKS)
        def _(k):
            base = pl.multiple_of(wid * PER_W + k * CH, CH)
            pltpu.sync_copy(gidx_hbm.at[pl.ds(base, CH)], gi_v)
            pltpu.sync_copy(sidx_hbm.at[pl.ds(base, CH)], si_v)
            pltpu.sync_copy(vals_hbm.at[pl.ds(base, CH)], va_v)
            pltpu.async_copy(table_hbm.at[gi_v], rows_v, sem).wait()

            @pl.loop(0, CH)
            def _(j):
                scale = plsc.load_gather(va_v, [jnp.full((16,), j, jnp.int32)])
                for g in range(D // 16):
                    sl = pl.ds(g * 16, 16)
                    rows_v[j, sl] = rows_v[j, sl] * scale

            pltpu.sync_copy(rows_v, acc.at[si_v], add=True)

        plsc.subcore_barrier()

        @pl.loop(sid, N_ZCH, step=NUM_SUBCORES)
        def _(k):
            base = pl.multiple_of(k * ZCH, ZCH)
            sl = pl.ds(base, ZCH)
            pltpu.sync_copy(acc.at[sl, :], out_hbm.at[cid, sl, :])

    return phase


_phase_w = _sc_phase(D)
_phase_x = _sc_phase(N)


def _tc_combine(relu: bool):
    bn = 1000

    def body(p_ref, o_ref):
        s = p_ref[0] + p_ref[1]
        if relu:
            s = jnp.maximum(s, 0.0)
        o_ref[...] = s

    return pl.pallas_call(
        body,
        grid=(N // bn,),
        in_specs=[pl.BlockSpec((NUM_CORES, bn, D), lambda i: (0, i, 0))],
        out_specs=pl.BlockSpec((bn, D), lambda i: (i, 0)),
        out_shape=jax.ShapeDtypeStruct((N, D), jnp.float32),
    )


_combine_sum = _tc_combine(relu=False)
_combine_relu = _tc_combine(relu=True)


def kernel(feat_rows, feat_cols, feat_vals, edge_index, adj_vals, weights):
    src = edge_index[1]
    dst = edge_index[0]
    pa = _phase_w(weights, feat_cols, feat_rows, feat_vals)
    xw = _combine_sum(pa)
    pb = _phase_x(xw, src, dst, adj_vals)
    return _combine_relu(pb)


# speedup vs baseline: 7.3614x; 2.1404x over previous
import dataclasses
import functools

import jax
import jax.numpy as jnp
from jax import lax
from jax.experimental import pallas as pl
from jax.experimental.pallas import tpu as pltpu
from jax.experimental.pallas import tpu_sc as plsc

N = 10000
D = 128
NNZ = 320000

NUM_CORES = 2
NUM_SUBCORES = 16
NW = NUM_CORES * NUM_SUBCORES
PER_W = NNZ // NW
CH = 40
NCH = PER_W // CH
NG = D // 16


def _sc_phase(table_rows: int, stage_table: bool):
    mesh = plsc.VectorSubcoreMesh(core_axis_name="c", subcore_axis_name="s")
    cp = pltpu.CompilerParams()
    if "needs_layout_passes" in pltpu.CompilerParams.__dataclass_fields__:
        cp = dataclasses.replace(cp, needs_layout_passes=False)

    scratch = [
        pltpu.VMEM_SHARED((N, D), jnp.float32),
        pltpu.VMEM((PER_W,), jnp.int32),
        pltpu.VMEM((PER_W,), jnp.int32),
        pltpu.VMEM((PER_W,), jnp.float32),
        pltpu.VMEM((CH, D), jnp.float32),
        pltpu.VMEM((CH, D), jnp.float32),
        pltpu.VMEM((CH, D), jnp.float32),
        pltpu.VMEM((CH, D), jnp.float32),
        pltpu.SemaphoreType.DMA,
        pltpu.SemaphoreType.DMA,
        pltpu.SemaphoreType.DMA,
        pltpu.SemaphoreType.DMA,
    ]
    if stage_table:
        scratch.append(pltpu.VMEM((table_rows, D), jnp.float32))

    @functools.partial(
        pl.kernel,
        out_type=jax.ShapeDtypeStruct((NUM_CORES, N, D), jnp.float32),
        mesh=mesh,
        compiler_params=cp,
        scratch_types=scratch,
    )
    def phase(table_hbm, gidx_hbm, sidx_hbm, vals_hbm, out_hbm,
              acc, gi_v, si_v, va_v, gb0, gb1, sb0, sb1,
              gsem0, gsem1, ssem0, ssem1, *maybe_tbl):
        cid = lax.axis_index("c")
        sid = lax.axis_index("s")
        wid = cid * NUM_SUBCORES + sid
        tbl = maybe_tbl[0] if stage_table else table_hbm

        base0 = pl.multiple_of(wid * PER_W, PER_W)
        pltpu.sync_copy(gidx_hbm.at[pl.ds(base0, PER_W)], gi_v)
        pltpu.sync_copy(sidx_hbm.at[pl.ds(base0, PER_W)], si_v)
        pltpu.sync_copy(vals_hbm.at[pl.ds(base0, PER_W)], va_v)

        if stage_table:
            pltpu.sync_copy(table_hbm, tbl)

        zero16 = jnp.zeros((16,), jnp.float32)

        @pl.loop(0, CH)
        def _(j):
            for g in range(NG):
                sb0[j, pl.ds(g * 16, 16)] = zero16

        @pl.loop(sid, N // CH, step=NUM_SUBCORES)
        def _(k):
            base = pl.multiple_of(k * CH, CH)
            pltpu.sync_copy(sb0, acc.at[pl.ds(base, CH), :])

        plsc.subcore_barrier()

        gbufs = (gb0, gb1)
        sbufs = (sb0, sb1)
        gsems = (gsem0, gsem1)
        ssems = (ssem0, ssem1)

        def gather_cp(c, b, sem):
            return pltpu.make_async_copy(
                tbl.at[gi_v.at[pl.ds(c * CH, CH)]], b, sem)

        def scatter_cp(c, b, sem):
            return pltpu.make_async_copy(
                b, acc.at[si_v.at[pl.ds(c * CH, CH)]], sem)

        gather_cp(0, gb0, gsem0).start()
        gather_cp(1, gb1, gsem1).start()

        @pl.loop(0, NCH // 2)
        def _(t):
            for b in range(2):
                c = 2 * t + b
                gather_cp(c, gbufs[b], gsems[b]).wait()

                @pl.when(t > 0)
                def _():
                    scatter_cp(c - 2, sbufs[b], ssems[b]).wait()

                @pl.loop(0, CH)
                def _(j):
                    scale = plsc.load_gather(
                        va_v, [jnp.full((16,), c * CH + j, jnp.int32)])
                    for g in range(NG):
                        sl = pl.ds(g * 16, 16)
                        sbufs[b][j, sl] = gbufs[b][j, sl] * scale

                pltpu.async_copy(sbufs[b],
                                 acc.at[si_v.at[pl.ds(c * CH, CH)]],
                                 ssems[b], add=True)

                @pl.when(c + 2 < NCH)
                def _():
                    gather_cp(c + 2, gbufs[b], gsems[b]).start()

        scatter_cp(NCH - 2, sb0, ssem0).wait()
        scatter_cp(NCH - 1, sb1, ssem1).wait()
        plsc.subcore_barrier()

        @pl.loop(sid, N // CH, step=NUM_SUBCORES)
        def _(k):
            base = pl.multiple_of(k * CH, CH)
            sl = pl.ds(base, CH)
            pltpu.sync_copy(acc.at[sl, :], out_hbm.at[cid, sl, :])

    return phase


_phase_w = _sc_phase(D, stage_table=False)
_phase_x = _sc_phase(N, stage_table=False)


def _tc_combine(relu: bool):
    bn = 1000

    def body(p_ref, o_ref):
        s = p_ref[0] + p_ref[1]
        if relu:
            s = jnp.maximum(s, 0.0)
        o_ref[...] = s

    return pl.pallas_call(
        body,
        grid=(N // bn,),
        in_specs=[pl.BlockSpec((NUM_CORES, bn, D), lambda i: (0, i, 0))],
        out_specs=pl.BlockSpec((bn, D), lambda i: (i, 0)),
        out_shape=jax.ShapeDtypeStruct((N, D), jnp.float32),
    )


_combine_sum = _tc_combine(relu=False)
_combine_relu = _tc_combine(relu=True)


def kernel(feat_rows, feat_cols, feat_vals, edge_index, adj_vals, weights):
    src = edge_index[1]
    dst = edge_index[0]
    pa = _phase_w(weights, feat_cols, feat_rows, feat_vals)
    xw = _combine_sum(pa)
    pb = _phase_x(xw, src, dst, adj_vals)
    return _combine_relu(pb)
